# baseline (speedup 1.0000x reference)
import functools

import jax
import jax.numpy as jnp
from jax import lax
from jax.experimental import pallas as pl
from jax.experimental.pallas import tpu as pltpu
from jax.experimental.pallas import tpu_sc as plsc

DIM = 512
CB = 256
NCB = 16
DPC = 32
TBLK = 1024
NCHUNK = 4
NW = 32


def _logits_kernel(x_ref, w_ref, b_ref, idx_ref):
    xbf = x_ref[...].astype(jnp.bfloat16)
    cols = []
    for c in range(NCB):
        wc = w_ref[c * DPC:(c + 1) * DPC, :]
        logits = jnp.dot(xbf[:, c * DPC:(c + 1) * DPC], wc,
                         preferred_element_type=jnp.float32)
        logits = logits + b_ref[c:c + 1, :]
        k = jnp.argmax(logits, axis=1).astype(jnp.int32) + c * CB
        cols.append(k[:, None])
    idx_ref[...] = jnp.concatenate(cols, axis=1)


def _err_kernel(x_ref, r_ref, out_ref):
    i = pl.program_id(0)

    @pl.when(i == 0)
    def _init():
        out_ref[0] = 0.0
        out_ref[1] = 0.0

    xall = x_ref[...]
    d = r_ref[...] - xall
    out_ref[0] += jnp.sum(d * d)
    out_ref[1] += jnp.sum(xall * xall)


def _make_sc_gather(chunk_rows):
    bpw = chunk_rows // NW
    mesh = plsc.VectorSubcoreMesh(core_axis_name="c", subcore_axis_name="s")

    @functools.partial(
        pl.kernel, mesh=mesh,
        compiler_params=pltpu.CompilerParams(use_tc_tiling_on_sc=False),
        out_type=jax.ShapeDtypeStruct((chunk_rows, DPC), jnp.float32),
        scratch_types=[
            pltpu.VMEM((bpw,), jnp.int32),
            pltpu.VMEM((bpw, DPC), jnp.float32),
            pltpu.SemaphoreType.DMA,
        ],
    )
    def sc_gather(table_hbm, idx_hbm, out_hbm, idx_v, rows_v, sem):
        wid = lax.axis_index("s") * 2 + lax.axis_index("c")
        base = wid * bpw
        pltpu.sync_copy(idx_hbm.at[pl.ds(base, bpw)], idx_v)
        pltpu.async_copy(table_hbm.at[idx_v], rows_v, sem).wait()
        pltpu.sync_copy(rows_v, out_hbm.at[pl.ds(base, bpw)])

    return sc_gather


def kernel(x, W, b, to_output, mask):
    del mask
    n_tokens = x.shape[0]
    chunk_tokens = n_tokens // NCHUNK
    chunk_rows = chunk_tokens * NCB

    w4 = W.reshape(NCB, CB, NCB, DPC)
    t4 = to_output.reshape(NCB, CB, NCB, DPC)
    diag = jnp.arange(NCB)
    wt = jnp.transpose(w4[diag, :, diag, :], (0, 2, 1))
    wt = wt.reshape(NCB * DPC, CB).astype(jnp.bfloat16)
    table = t4[diag, :, diag, :].reshape(NCB * CB, DPC)
    b2 = b.reshape(NCB, CB)

    tc_logits = functools.partial(
        pl.pallas_call,
        _logits_kernel,
        grid=(chunk_tokens // TBLK,),
        in_specs=[
            pl.BlockSpec((TBLK, DIM), lambda i: (i, 0)),
            pl.BlockSpec((NCB * DPC, CB), lambda i: (0, 0)),
            pl.BlockSpec((NCB, CB), lambda i: (0, 0)),
        ],
        out_specs=pl.BlockSpec((TBLK, NCB), lambda i: (i, 0)),
        out_shape=jax.ShapeDtypeStruct((chunk_tokens, NCB), jnp.int32),
    )()
    tc_err = functools.partial(
        pl.pallas_call,
        _err_kernel,
        grid=(chunk_tokens // TBLK,),
        in_specs=[
            pl.BlockSpec((TBLK, DIM), lambda i: (i, 0)),
            pl.BlockSpec((TBLK, DIM), lambda i: (i, 0)),
        ],
        out_specs=pl.BlockSpec(memory_space=pltpu.SMEM),
        out_shape=jax.ShapeDtypeStruct((2,), jnp.float32),
    )()
    sc_gather = _make_sc_gather(chunk_rows)

    err_sum = jnp.float32(0.0)
    x_sum = jnp.float32(0.0)
    for k in range(NCHUNK):
        xk = x[k * chunk_tokens:(k + 1) * chunk_tokens]
        idx = tc_logits(xk, wt, b2)
        recon = sc_gather(table, idx.reshape(chunk_rows))
        part = tc_err(xk, recon.reshape(chunk_tokens, DIM))
        err_sum = err_sum + part[0]
        x_sum = x_sum + part[1]
    return err_sum / (x_sum + 1e-20)


# speedup vs baseline: 1.0348x; 1.0348x over previous
import functools

import jax
import jax.numpy as jnp
from jax import lax
from jax.experimental import pallas as pl
from jax.experimental.pallas import tpu as pltpu
from jax.experimental.pallas import tpu_sc as plsc

DIM = 512
CB = 256
NCB = 16
DPC = 32
TBLK = 1024
NCHUNK = 1
NW = 32


def _logits_kernel(x_ref, w_ref, b_ref, idx_ref):
    xbf = x_ref[...].astype(jnp.bfloat16)
    cols = []
    for c in range(NCB):
        wc = w_ref[c * DPC:(c + 1) * DPC, :]
        logits = jnp.dot(xbf[:, c * DPC:(c + 1) * DPC], wc,
                         preferred_element_type=jnp.float32)
        logits = logits + b_ref[c:c + 1, :]
        k = jnp.argmax(logits, axis=1).astype(jnp.int32) + c * CB
        cols.append(k[:, None])
    idx_ref[...] = jnp.concatenate(cols, axis=1)


def _err_kernel(x_ref, r_ref, out_ref):
    i = pl.program_id(0)

    @pl.when(i == 0)
    def _init():
        out_ref[0] = 0.0
        out_ref[1] = 0.0

    xall = x_ref[...]
    d = r_ref[...] - xall
    out_ref[0] += jnp.sum(d * d)
    out_ref[1] += jnp.sum(xall * xall)


def _make_sc_gather(chunk_rows):
    bpw = chunk_rows // NW
    sub = min(bpw, 2048)
    nsub = bpw // sub
    mesh = plsc.VectorSubcoreMesh(core_axis_name="c", subcore_axis_name="s")

    @functools.partial(
        pl.kernel, mesh=mesh,
        compiler_params=pltpu.CompilerParams(use_tc_tiling_on_sc=False),
        out_type=jax.ShapeDtypeStruct((chunk_rows, DPC), jnp.float32),
        scratch_types=[
            pltpu.VMEM((bpw,), jnp.int32),
            pltpu.VMEM((sub, DPC), jnp.float32),
            pltpu.SemaphoreType.DMA,
        ],
    )
    def sc_gather(table_hbm, idx_hbm, out_hbm, idx_v, rows_v, sem):
        wid = lax.axis_index("s") * 2 + lax.axis_index("c")
        base = wid * bpw
        pltpu.sync_copy(idx_hbm.at[pl.ds(base, bpw)], idx_v)
        for j in range(nsub):
            pltpu.async_copy(table_hbm.at[idx_v.at[pl.ds(j * sub, sub)]],
                             rows_v, sem).wait()
            pltpu.sync_copy(rows_v, out_hbm.at[pl.ds(base + j * sub, sub)])

    return sc_gather


def kernel(x, W, b, to_output, mask):
    del mask
    n_tokens = x.shape[0]
    chunk_tokens = n_tokens // NCHUNK
    chunk_rows = chunk_tokens * NCB

    w4 = W.reshape(NCB, CB, NCB, DPC)
    t4 = to_output.reshape(NCB, CB, NCB, DPC)
    diag = jnp.arange(NCB)
    wt = jnp.transpose(w4[diag, :, diag, :], (0, 2, 1))
    wt = wt.reshape(NCB * DPC, CB).astype(jnp.bfloat16)
    table = t4[diag, :, diag, :].reshape(NCB * CB, DPC)
    b2 = b.reshape(NCB, CB)

    tc_logits = functools.partial(
        pl.pallas_call,
        _logits_kernel,
        grid=(chunk_tokens // TBLK,),
        in_specs=[
            pl.BlockSpec((TBLK, DIM), lambda i: (i, 0)),
            pl.BlockSpec((NCB * DPC, CB), lambda i: (0, 0)),
            pl.BlockSpec((NCB, CB), lambda i: (0, 0)),
        ],
        out_specs=pl.BlockSpec((TBLK, NCB), lambda i: (i, 0)),
        out_shape=jax.ShapeDtypeStruct((chunk_tokens, NCB), jnp.int32),
    )()
    tc_err = functools.partial(
        pl.pallas_call,
        _err_kernel,
        grid=(chunk_tokens // TBLK,),
        in_specs=[
            pl.BlockSpec((TBLK, DIM), lambda i: (i, 0)),
            pl.BlockSpec((TBLK, DIM), lambda i: (i, 0)),
        ],
        out_specs=pl.BlockSpec(memory_space=pltpu.SMEM),
        out_shape=jax.ShapeDtypeStruct((2,), jnp.float32),
    )()
    sc_gather = _make_sc_gather(chunk_rows)

    err_sum = jnp.float32(0.0)
    x_sum = jnp.float32(0.0)
    for k in range(NCHUNK):
        xk = x[k * chunk_tokens:(k + 1) * chunk_tokens]
        idx = tc_logits(xk, wt, b2)
        recon = sc_gather(table, idx.reshape(chunk_rows))
        part = tc_err(xk, recon.reshape(chunk_tokens, DIM))
        err_sum = err_sum + part[0]
        x_sum = x_sum + part[1]
    return err_sum / (x_sum + 1e-20)


# speedup vs baseline: 3.4298x; 3.3146x over previous
import jax
import jax.numpy as jnp
from jax.experimental import pallas as pl
from jax.experimental.pallas import tpu as pltpu

DIM = 512
CB = 256
NCB = 16
DPC = 32
TBLK = 1024


def _vq_kernel(x_ref, a_ref, b_ref, out_ref, nrm_ref, acc_ref):
    i = pl.program_id(0)

    @pl.when(i == 0)
    def _init():
        acc_ref[0] = 0.0
        acc_ref[1] = 0.0
        for c in range(NCB):
            tt = a_ref[c * DPC:(c + 1) * DPC, CB:].astype(jnp.float32)
            nrm_ref[c:c + 1, :] = 0.25 * jnp.sum(tt * tt, axis=0,
                                                 keepdims=True)

    xall = x_ref[...]
    xbf = xall.astype(jnp.bfloat16)
    vaccs = [jnp.zeros((TBLK, CB), jnp.float32) for _ in range(4)]
    for c in range(NCB):
        ac = a_ref[c * DPC:(c + 1) * DPC, :]
        prod = jnp.dot(xbf[:, c * DPC:(c + 1) * DPC], ac,
                       preferred_element_type=jnp.float32)
        logits = prod[:, :CB] + b_ref[c:c + 1, :]
        m = jnp.max(logits, axis=1, keepdims=True)
        fval = prod[:, CB:] + nrm_ref[c:c + 1, :]
        vaccs[c % 4] += jnp.where(logits == m, fval, 0.0)
    vacc = (vaccs[0] + vaccs[1]) + (vaccs[2] + vaccs[3])
    acc_ref[0] += jnp.sum(vacc)
    acc_ref[1] += jnp.sum(xall * xall)

    @pl.when(i == pl.num_programs(0) - 1)
    def _fin():
        s = acc_ref[1] + 1e-20
        out_ref[...] = jnp.full((1, 1), (acc_ref[0] + acc_ref[1]) / s,
                                dtype=jnp.float32)


def kernel(x, W, b, to_output, mask):
    del mask
    n_tokens = x.shape[0]

    w4 = W.reshape(NCB, CB, NCB, DPC)
    t4 = to_output.reshape(NCB, CB, NCB, DPC)
    diag = jnp.arange(NCB)
    wblk = w4[diag, :, diag, :]
    tblk = t4[diag, :, diag, :]
    a = jnp.concatenate(
        [jnp.transpose(wblk, (0, 2, 1)).reshape(NCB * DPC, CB),
         -2.0 * jnp.transpose(tblk, (0, 2, 1)).reshape(NCB * DPC, CB)],
        axis=1).astype(jnp.bfloat16)
    b2 = b.reshape(NCB, CB)

    grid = n_tokens // TBLK
    out = pl.pallas_call(
        _vq_kernel,
        grid=(grid,),
        in_specs=[
            pl.BlockSpec((TBLK, DIM), lambda i: (i, 0)),
            pl.BlockSpec((NCB * DPC, 2 * CB), lambda i: (0, 0)),
            pl.BlockSpec((NCB, CB), lambda i: (0, 0)),
        ],
        out_specs=pl.BlockSpec((1, 1), lambda i: (0, 0)),
        out_shape=jax.ShapeDtypeStruct((1, 1), jnp.float32),
        scratch_shapes=[pltpu.VMEM((NCB, CB), jnp.float32),
                        pltpu.SMEM((2,), jnp.float32)],
    )(x, a, b2)
    return out.reshape(())
